# baseline (speedup 1.0000x reference)
import functools

import jax
import jax.numpy as jnp
from jax.experimental import pallas as pl

HIDDEN = 4096
NUM_EXPERTS = 64
TOKENS = 32768
BLK = 512


def _router_kernel(x_ref, w_ref, b_ref, logits_ref, probs_ref, idx_ref):
    x = x_ref[...]
    w = w_ref[...]
    logits = jax.lax.dot_general(
        x, w, (((1,), (1,)), ((), ())), preferred_element_type=jnp.float32
    )
    logits = logits + b_ref[...]
    logits_ref[...] = logits

    ids = jax.lax.broadcasted_iota(jnp.int32, logits.shape, 1)
    m1 = jnp.max(logits, axis=-1, keepdims=True)
    i1 = jnp.argmax(logits, axis=-1, keepdims=True).astype(jnp.int32)
    masked = jnp.where(ids == i1, -jnp.inf, logits)
    m2 = jnp.max(masked, axis=-1, keepdims=True)
    i2 = jnp.argmax(masked, axis=-1, keepdims=True).astype(jnp.int32)
    e2 = jnp.exp(m2 - m1)
    denom = 1.0 + e2
    probs_ref[:, 0:1] = 1.0 / denom
    probs_ref[:, 1:2] = e2 / denom
    idx_ref[:, 0:1] = i1
    idx_ref[:, 1:2] = i2


@jax.jit
def kernel(x, W, b):
    grid = (TOKENS // BLK,)
    out = pl.pallas_call(
        _router_kernel,
        grid=grid,
        in_specs=[
            pl.BlockSpec((BLK, HIDDEN), lambda i: (i, 0)),
            pl.BlockSpec((NUM_EXPERTS, HIDDEN), lambda i: (0, 0)),
            pl.BlockSpec((1, NUM_EXPERTS), lambda i: (0, 0)),
        ],
        out_specs=[
            pl.BlockSpec((BLK, NUM_EXPERTS), lambda i: (i, 0)),
            pl.BlockSpec((BLK, 2), lambda i: (i, 0)),
            pl.BlockSpec((BLK, 2), lambda i: (i, 0)),
        ],
        out_shape=[
            jax.ShapeDtypeStruct((TOKENS, NUM_EXPERTS), jnp.float32),
            jax.ShapeDtypeStruct((TOKENS, 2), jnp.float32),
            jax.ShapeDtypeStruct((TOKENS, 2), jnp.int32),
        ],
    )(x, W.reshape(NUM_EXPERTS, HIDDEN), b.reshape(1, NUM_EXPERTS))
    logits, probs, idx = out
    return (probs, idx, logits)


# speedup vs baseline: 1.0162x; 1.0162x over previous
import functools

import jax
import jax.numpy as jnp
from jax.experimental import pallas as pl
from jax.experimental.pallas import tpu as pltpu

HIDDEN = 4096
NUM_EXPERTS = 64
TOKENS = 32768
BLK = 1024


def _router_kernel(x_ref, w_ref, b_ref, logits_ref, probs_ref, idx_ref):
    x = x_ref[...]
    w = w_ref[...]
    logits = jax.lax.dot_general(
        x, w, (((1,), (1,)), ((), ())), preferred_element_type=jnp.float32
    )
    logits = logits + b_ref[...]
    logits_ref[...] = logits

    ids = jax.lax.broadcasted_iota(jnp.int32, logits.shape, 1)
    m1 = jnp.max(logits, axis=-1, keepdims=True)
    i1 = jnp.argmax(logits, axis=-1, keepdims=True).astype(jnp.int32)
    masked = jnp.where(ids == i1, -jnp.inf, logits)
    m2 = jnp.max(masked, axis=-1, keepdims=True)
    i2 = jnp.argmax(masked, axis=-1, keepdims=True).astype(jnp.int32)
    e2 = jnp.exp(m2 - m1)
    denom = 1.0 + e2
    probs_ref[:, 0:1] = 1.0 / denom
    probs_ref[:, 1:2] = e2 / denom
    idx_ref[:, 0:1] = i1
    idx_ref[:, 1:2] = i2


@jax.jit
def kernel(x, W, b):
    grid = (TOKENS // BLK,)
    out = pl.pallas_call(
        _router_kernel,
        grid=grid,
        in_specs=[
            pl.BlockSpec((BLK, HIDDEN), lambda i: (i, 0)),
            pl.BlockSpec((NUM_EXPERTS, HIDDEN), lambda i: (0, 0)),
            pl.BlockSpec((1, NUM_EXPERTS), lambda i: (0, 0)),
        ],
        out_specs=[
            pl.BlockSpec((BLK, NUM_EXPERTS), lambda i: (i, 0)),
            pl.BlockSpec((BLK, 2), lambda i: (i, 0)),
            pl.BlockSpec((BLK, 2), lambda i: (i, 0)),
        ],
        out_shape=[
            jax.ShapeDtypeStruct((TOKENS, NUM_EXPERTS), jnp.float32),
            jax.ShapeDtypeStruct((TOKENS, 2), jnp.float32),
            jax.ShapeDtypeStruct((TOKENS, 2), jnp.int32),
        ],
    )(x, W.reshape(NUM_EXPERTS, HIDDEN), b.reshape(1, NUM_EXPERTS))
    logits, probs, idx = out
    return (probs, idx, logits)


# speedup vs baseline: 1.0372x; 1.0207x over previous
import jax
import jax.numpy as jnp
from jax.experimental import pallas as pl
from jax.experimental.pallas import tpu as pltpu

HIDDEN = 4096
NUM_EXPERTS = 64
TOKENS = 32768
BLK = 1024
KSPLIT = HIDDEN // 2


def _router_kernel(xa_ref, xb_ref, wa_ref, wb_ref, b_ref,
                   logits_ref, probs_ref, idx_ref):
    dn = (((1,), (1,)), ((), ()))
    logits = jax.lax.dot_general(
        xa_ref[...], wa_ref[...], dn, preferred_element_type=jnp.float32
    )
    logits = logits + jax.lax.dot_general(
        xb_ref[...], wb_ref[...], dn, preferred_element_type=jnp.float32
    )
    logits = logits + b_ref[...]
    logits_ref[...] = logits

    ids = jax.lax.broadcasted_iota(jnp.int32, logits.shape, 1)
    m1 = jnp.max(logits, axis=-1, keepdims=True)
    i1 = jnp.argmax(logits, axis=-1, keepdims=True).astype(jnp.int32)
    masked = jnp.where(ids == i1, -jnp.inf, logits)
    m2 = jnp.max(masked, axis=-1, keepdims=True)
    i2 = jnp.argmax(masked, axis=-1, keepdims=True).astype(jnp.int32)
    e2 = jnp.exp(m2 - m1)
    denom = 1.0 + e2
    probs_ref[:, 0:1] = 1.0 / denom
    probs_ref[:, 1:2] = e2 / denom
    idx_ref[:, 0:1] = i1
    idx_ref[:, 1:2] = i2


@jax.jit
def kernel(x, W, b):
    grid = (TOKENS // BLK,)
    out = pl.pallas_call(
        _router_kernel,
        grid=grid,
        in_specs=[
            pl.BlockSpec((BLK, KSPLIT), lambda i: (i, 0)),
            pl.BlockSpec((BLK, KSPLIT), lambda i: (i, 1)),
            pl.BlockSpec((NUM_EXPERTS, KSPLIT), lambda i: (0, 0)),
            pl.BlockSpec((NUM_EXPERTS, KSPLIT), lambda i: (0, 1)),
            pl.BlockSpec((1, NUM_EXPERTS), lambda i: (0, 0)),
        ],
        out_specs=[
            pl.BlockSpec((BLK, NUM_EXPERTS), lambda i: (i, 0)),
            pl.BlockSpec((BLK, 2), lambda i: (i, 0)),
            pl.BlockSpec((BLK, 2), lambda i: (i, 0)),
        ],
        out_shape=[
            jax.ShapeDtypeStruct((TOKENS, NUM_EXPERTS), jnp.float32),
            jax.ShapeDtypeStruct((TOKENS, 2), jnp.float32),
            jax.ShapeDtypeStruct((TOKENS, 2), jnp.int32),
        ],
    )(x, x, W, W, b.reshape(1, NUM_EXPERTS))
    logits, probs, idx = out
    return (probs, idx, logits)
